# Initial kernel scaffold; baseline (speedup 1.0000x reference)
import jax
import jax.numpy as jnp
from jax.experimental import pallas as pl


def kernel(gt_boxes, feature_select_weight, feature_maps_shape):
    raise NotImplementedError("write your pallas kernel here")


# speedup vs baseline: 9.1492x; 9.1492x over previous
import jax
import jax.numpy as jnp
from jax import lax
from jax.experimental import pallas as pl
from jax.experimental.pallas import tpu as pltpu
from jax.experimental.pallas import tpu_sc as plsc

_NUM_CLS = 80
_CLS_C = _NUM_CLS + 2
_REG_C = 6
_L = 16
_NB = 32
_STRIDES = (8, 16, 32, 64, 128)
_FEATURE_SHAPES = ((64, 64), (32, 32), (16, 16), (8, 8), (4, 4))
_NLOC = sum(h * w for h, w in _FEATURE_SHAPES)
_NCHUNK = _NLOC // _L
_NW = 32
_JMAX = -(-_NCHUNK // _NW)
_LVL_CHUNK_START = (0, 256, 320, 336, 340)


def _tec_body(gt_hbm, fsw_hbm, fms_hbm, cls_hbm, reg_hbm,
              gt_v, fsw_v, fms_v, clsb_v, regb_v):
    nc = 2
    wid = lax.axis_index("s") * nc + lax.axis_index("c")

    pltpu.sync_copy(gt_hbm, gt_v)
    pltpu.sync_copy(fsw_hbm, fsw_v)
    pltpu.sync_copy(fms_hbm, fms_v)

    f32 = jnp.float32
    i32 = jnp.int32
    iota = jnp.arange(_L, dtype=i32)
    zeros = jnp.zeros((_L,), f32)

    x1v, y1v, x2v, y2v = [], [], [], []
    a1xv, a2xv, a1yv, a2yv, labv, vldv = [], [], [], [], [], []
    for h in range(2):
        s = pl.ds(h * _L, _L)
        x1 = gt_v[0, s]
        y1 = gt_v[1, s]
        x2 = gt_v[2, s]
        y2 = gt_v[3, s]
        cxv = (x1 + x2) * f32(0.5)
        cyv = (y1 + y2) * f32(0.5)
        swv = (x2 - x1) * f32(0.2)
        shv = (y2 - y1) * f32(0.2)
        x1v.append(x1)
        y1v.append(y1)
        x2v.append(x2)
        y2v.append(y2)
        a1xv.append(cxv - swv * f32(0.5))
        a2xv.append(cxv + swv * f32(0.5))
        a1yv.append(cyv - shv * f32(0.5))
        a2yv.append(cyv + shv * f32(0.5))
        labv.append(gt_v[4, s])
        vldv.append(jnp.abs(x1) + jnp.abs(y1) + jnp.abs(x2) + jnp.abs(y2))

    fms_vec = fms_v[...]

    for i in range(_CLS_C):
        clsb_v[pl.ds(i * _L, _L)] = zeros

    lane_cls = iota * i32(_CLS_C)
    idx80 = lane_cls + i32(_NUM_CLS)
    idx81 = lane_cls + i32(_NUM_CLS + 1)
    lane_reg = iota * i32(_REG_C)

    def chunk_body(j, carry):
        c = wid + _NW * j

        @pl.when(c < _NCHUNK)
        def _():
            ge1 = c >= _LVL_CHUNK_START[1]
            ge2 = c >= _LVL_CHUNK_START[2]
            ge3 = c >= _LVL_CHUNK_START[3]
            ge4 = c >= _LVL_CHUNK_START[4]

            def chain(v4, v3, v2, v1, v0):
                return jnp.where(ge4, v4, jnp.where(ge3, v3, jnp.where(
                    ge2, v2, jnp.where(ge1, v1, v0))))

            stride_f = chain(*[f32(_STRIDES[l]) for l in (4, 3, 2, 1, 0)])
            inv4s = chain(*[f32(0.25 / _STRIDES[l]) for l in (4, 3, 2, 1, 0)])
            fw_m1 = chain(*[i32(_FEATURE_SHAPES[l][1] - 1) for l in (4, 3, 2, 1, 0)])
            shift = chain(*[i32(6 - l) for l in (4, 3, 2, 1, 0)])
            cbase = chain(*[i32(_LVL_CHUNK_START[l]) for l in (4, 3, 2, 1, 0)])
            fht = chain(*[fms_vec[2 * l] for l in (4, 3, 2, 1, 0)])
            fwt = chain(*[fms_vec[2 * l + 1] for l in (4, 3, 2, 1, 0)])

            lwv = [chain(fsw_v[4, pl.ds(h * _L, _L)], fsw_v[3, pl.ds(h * _L, _L)],
                         fsw_v[2, pl.ds(h * _L, _L)], fsw_v[1, pl.ds(h * _L, _L)],
                         fsw_v[0, pl.ds(h * _L, _L)]) for h in range(2)]

            li = (c - cbase) * i32(_L) + iota
            gx = jnp.bitwise_and(li, fw_m1)
            gy = jnp.right_shift(li, shift)
            gxf = gx.astype(f32)
            gyf = gy.astype(f32)
            sx = (gxf + f32(0.5)) * stride_f
            sy = (gyf + f32(0.5)) * stride_f
            xg0 = gxf * stride_f
            xg1 = xg0 + stride_f
            yg0 = gyf * stride_f
            yg1 = yg0 + stride_f
            m_xlo = gx >= fwt - 1
            m_xlt1 = gx < 1
            m_xfwt = gx < fwt
            m_ylo = gy >= fht - 1
            m_ylt1 = gy < 1
            m_yfht = gy < fht

            best = jnp.full((_L,), 2.0e7, f32)
            wdl = zeros
            wdt = zeros
            wdr = zeros
            wdb = zeros
            wposf = zeros
            wlab = zeros
            wlw = zeros
            for b in range(_NB):
                h, ln = b // _L, b % _L
                x1 = x1v[h][ln]
                y1 = y1v[h][ln]
                x2 = x2v[h][ln]
                y2 = y2v[h][ln]
                a1x = a1xv[h][ln]
                a2x = a2xv[h][ln]
                a1y = a1yv[h][ln]
                a2y = a2yv[h][ln]
                lab = labv[h][ln]
                vld = vldv[h][ln]
                lw = lwv[h][ln]
                dl = jnp.maximum(sx - x1, f32(0))
                dt = jnp.maximum(sy - y1, f32(0))
                dr = jnp.maximum(x2 - sx, f32(0))
                db = jnp.maximum(y2 - sy, f32(0))
                inx = ((xg1 > a1x) | m_xlo) & (((xg0 < a2x) & m_xfwt) | m_xlt1)
                iny = ((yg1 > a1y) | m_ylo) & (((yg0 < a2y) & m_yfht) | m_ylt1)
                pos = inx & iny & (vld > f32(0))
                posf = jnp.where(pos, f32(1.0), f32(0.0))
                area = (dl + dr) * (dt + db)
                am = jnp.where(pos, area, f32(1.0e7))
                upd = am < best
                best = jnp.where(upd, am, best)
                wdl = jnp.where(upd, dl, wdl)
                wdt = jnp.where(upd, dt, wdt)
                wdr = jnp.where(upd, dr, wdr)
                wdb = jnp.where(upd, db, wdb)
                wposf = jnp.where(upd, posf, wposf)
                wlab = jnp.where(upd, lab, wlab)
                wlw = jnp.where(upd, lw, wlw)

            eps = f32(1e-7)
            ap = (jnp.minimum(wdl, wdr) * jnp.minimum(wdt, wdb)
                  / jnp.maximum(jnp.maximum(wdl, wdr), eps)
                  / jnp.maximum(jnp.maximum(wdt, wdb), eps))
            soft = jnp.where(wposf > f32(0.5), ap * wlw, f32(1.0))

            idx_cls = lane_cls + wlab.astype(i32)
            plsc.store_scatter(clsb_v, [idx_cls], wposf)
            plsc.store_scatter(clsb_v, [idx80], soft)
            plsc.store_scatter(clsb_v, [idx81], wposf)

            plsc.store_scatter(regb_v, [lane_reg], wdl * inv4s * wposf)
            plsc.store_scatter(regb_v, [lane_reg + 1], wdt * inv4s * wposf)
            plsc.store_scatter(regb_v, [lane_reg + 2], wdr * inv4s * wposf)
            plsc.store_scatter(regb_v, [lane_reg + 3], wdb * inv4s * wposf)
            plsc.store_scatter(regb_v, [lane_reg + 4], soft)
            plsc.store_scatter(regb_v, [lane_reg + 5], wposf)

            pltpu.sync_copy(clsb_v, cls_hbm.at[pl.ds(c * (_L * _CLS_C), _L * _CLS_C)])
            pltpu.sync_copy(regb_v, reg_hbm.at[pl.ds(c * (_L * _REG_C), _L * _REG_C)])

            plsc.store_scatter(clsb_v, [idx_cls], zeros)

        return carry

    lax.fori_loop(0, _JMAX, chunk_body, 0)


@jax.jit
def kernel(gt_boxes, feature_select_weight, feature_maps_shape):
    gt_t = jnp.transpose(gt_boxes).astype(jnp.float32)
    fsw_t = jnp.transpose(feature_select_weight).astype(jnp.float32)
    fms_flat = jnp.concatenate(
        [feature_maps_shape.reshape(-1).astype(jnp.int32),
         jnp.zeros((6,), jnp.int32)])

    mesh = plsc.VectorSubcoreMesh(core_axis_name="c", subcore_axis_name="s")
    run = pl.kernel(
        _tec_body,
        out_type=[
            jax.ShapeDtypeStruct((_NLOC * _CLS_C,), jnp.float32),
            jax.ShapeDtypeStruct((_NLOC * _REG_C,), jnp.float32),
        ],
        mesh=mesh,
        compiler_params=pltpu.CompilerParams(needs_layout_passes=False),
        scratch_types=[
            pltpu.VMEM((5, _NB), jnp.float32),
            pltpu.VMEM((5, _NB), jnp.float32),
            pltpu.VMEM((16,), jnp.int32),
            pltpu.VMEM((_L * _CLS_C,), jnp.float32),
            pltpu.VMEM((_L * _REG_C,), jnp.float32),
        ],
    )
    cls_flat, reg_flat = run(gt_t, fsw_t, fms_flat)
    return cls_flat.reshape(_NLOC, _CLS_C), reg_flat.reshape(_NLOC, _REG_C)


# speedup vs baseline: 9.1508x; 1.0002x over previous
import jax
import jax.numpy as jnp
from jax import lax
from jax.experimental import pallas as pl
from jax.experimental.pallas import tpu as pltpu
from jax.experimental.pallas import tpu_sc as plsc

_NUM_CLS = 80
_CLS_C = _NUM_CLS + 2
_REG_C = 6
_L = 16
_NB = 32
_STRIDES = (8, 16, 32, 64, 128)
_FEATURE_SHAPES = ((64, 64), (32, 32), (16, 16), (8, 8), (4, 4))
_NLOC = sum(h * w for h, w in _FEATURE_SHAPES)
_NCHUNK = _NLOC // _L
_NW = 32
_JMAX = 11
_LVL_CHUNK_START = (0, 256, 320, 336, 340)
_CLS_W = _L * _CLS_C
_REG_W = _L * _REG_C


def _tec_body(pk_hbm, cls_hbm, reg_hbm, pk_v, clsb_v, regb_v):
    nc = 2
    wid = lax.axis_index("s") * nc + lax.axis_index("c")

    pltpu.sync_copy(pk_hbm, pk_v)

    f32 = jnp.float32
    i32 = jnp.int32
    inf = f32(jnp.inf)
    iota = jnp.arange(_L, dtype=i32)
    zeros = jnp.zeros((_L,), f32)

    @pl.when(wid < _NCHUNK // _JMAX)
    def _():
        x1v, y1v, x2v, y2v = [], [], [], []
        a1xv, a2xv, a1yv, a2yv = [], [], [], []
        for h in range(2):
            s = pl.ds(h * _L, _L)
            x1 = pk_v[0, s]
            y1 = pk_v[1, s]
            x2 = pk_v[2, s]
            y2 = pk_v[3, s]
            cxv = (x1 + x2) * f32(0.5)
            cyv = (y1 + y2) * f32(0.5)
            swv = (x2 - x1) * f32(0.2)
            shv = (y2 - y1) * f32(0.2)
            valid = (jnp.abs(x1) + jnp.abs(y1) + jnp.abs(x2) + jnp.abs(y2)) > f32(0)
            x1v.append(x1)
            y1v.append(y1)
            x2v.append(x2)
            y2v.append(y2)
            a1xv.append(cxv - swv * f32(0.5))
            a2xv.append(jnp.where(valid, cxv + swv * f32(0.5), -inf))
            a1yv.append(cyv - shv * f32(0.5))
            a2yv.append(jnp.where(valid, cyv + shv * f32(0.5), -inf))

        fmsrow = pk_v[10, pl.ds(0, _L)]

        for i in range(_JMAX * _CLS_C):
            clsb_v[pl.ds(i * _L, _L)] = zeros

        lane_cls = iota * i32(_CLS_C)
        lane_reg = iota * i32(_REG_C)

        def chunk_body(j, carry):
            c = wid * _JMAX + j
            ge1 = c >= _LVL_CHUNK_START[1]
            ge2 = c >= _LVL_CHUNK_START[2]
            ge3 = c >= _LVL_CHUNK_START[3]
            ge4 = c >= _LVL_CHUNK_START[4]

            def chain(v4, v3, v2, v1, v0):
                return jnp.where(ge4, v4, jnp.where(ge3, v3, jnp.where(
                    ge2, v2, jnp.where(ge1, v1, v0))))

            stride_f = chain(*[f32(_STRIDES[l]) for l in (4, 3, 2, 1, 0)])
            inv4s = chain(*[f32(0.25 / _STRIDES[l]) for l in (4, 3, 2, 1, 0)])
            fw_m1 = chain(*[i32(_FEATURE_SHAPES[l][1] - 1) for l in (4, 3, 2, 1, 0)])
            shift = chain(*[i32(6 - l) for l in (4, 3, 2, 1, 0)])
            cbase = chain(*[i32(_LVL_CHUNK_START[l]) for l in (4, 3, 2, 1, 0)])
            fht_f = chain(*[fmsrow[2 * l] for l in (4, 3, 2, 1, 0)])
            fwt_f = chain(*[fmsrow[2 * l + 1] for l in (4, 3, 2, 1, 0)])
            lvl = (ge1.astype(i32) + ge2.astype(i32)
                   + ge3.astype(i32) + ge4.astype(i32))

            li = (c - cbase) * i32(_L) + iota
            gx = jnp.bitwise_and(li, fw_m1)
            gy = jnp.right_shift(li, shift)
            gxf = gx.astype(f32)
            gyf = gy.astype(f32)
            sx = (gxf + f32(0.5)) * stride_f
            sy = (gyf + f32(0.5)) * stride_f
            xg0 = gxf * stride_f
            xg1 = xg0 + stride_f
            yg0 = gyf * stride_f
            yg1 = yg0 + stride_f
            x_lo = jnp.where(gxf >= fwt_f - f32(1), inf, xg1)
            x_hi = jnp.where(gxf < f32(1), -inf,
                             jnp.where(gxf < fwt_f, xg0, inf))
            y_lo = jnp.where(gyf >= fht_f - f32(1), inf, yg1)
            y_hi = jnp.where(gyf < f32(1), -inf,
                             jnp.where(gyf < fht_f, yg0, inf))

            best = jnp.full((_L,), 2.0e7, f32)
            wbidx = jnp.zeros((_L,), i32)
            for b in range(_NB):
                h, ln = b // _L, b % _L
                x1 = x1v[h][ln]
                y1 = y1v[h][ln]
                x2 = x2v[h][ln]
                y2 = y2v[h][ln]
                dl = jnp.maximum(sx - x1, f32(0))
                dt = jnp.maximum(sy - y1, f32(0))
                dr = jnp.maximum(x2 - sx, f32(0))
                db = jnp.maximum(y2 - sy, f32(0))
                pos = (((x_lo > a1xv[h][ln]) & (x_hi < a2xv[h][ln]))
                       & ((y_lo > a1yv[h][ln]) & (y_hi < a2yv[h][ln])))
                area = (dl + dr) * (dt + db)
                am = jnp.where(pos, area, f32(1.0e7))
                upd = am < best
                best = jnp.where(upd, am, best)
                wbidx = jnp.where(upd, i32(b), wbidx)

            rowz = jnp.zeros((_L,), i32)
            wx1 = plsc.load_gather(pk_v, [rowz, wbidx])
            wy1 = plsc.load_gather(pk_v, [rowz + 1, wbidx])
            wx2 = plsc.load_gather(pk_v, [rowz + 2, wbidx])
            wy2 = plsc.load_gather(pk_v, [rowz + 3, wbidx])
            wlab = plsc.load_gather(pk_v, [rowz + 4, wbidx])
            wlw = plsc.load_gather(pk_v, [rowz + 5 + lvl, wbidx])
            wdl = jnp.maximum(sx - wx1, f32(0))
            wdt = jnp.maximum(sy - wy1, f32(0))
            wdr = jnp.maximum(wx2 - sx, f32(0))
            wdb = jnp.maximum(wy2 - sy, f32(0))
            wpos = best < f32(1.0e7)
            wposf = jnp.where(wpos, f32(1.0), f32(0.0))

            eps = f32(1e-7)
            ap = (jnp.minimum(wdl, wdr) * jnp.minimum(wdt, wdb)
                  / jnp.maximum(jnp.maximum(wdl, wdr), eps)
                  / jnp.maximum(jnp.maximum(wdt, wdb), eps))
            soft = jnp.where(wpos, ap * wlw, f32(1.0))

            cls_base = j * i32(_CLS_W)
            idx_cls = cls_base + lane_cls
            plsc.store_scatter(clsb_v, [idx_cls + wlab.astype(i32)], wposf)
            plsc.store_scatter(clsb_v, [idx_cls + i32(_NUM_CLS)], soft)
            plsc.store_scatter(clsb_v, [idx_cls + i32(_NUM_CLS + 1)], wposf)

            reg_base = j * i32(_REG_W)
            idx_reg = reg_base + lane_reg
            plsc.store_scatter(regb_v, [idx_reg], wdl * inv4s * wposf)
            plsc.store_scatter(regb_v, [idx_reg + 1], wdt * inv4s * wposf)
            plsc.store_scatter(regb_v, [idx_reg + 2], wdr * inv4s * wposf)
            plsc.store_scatter(regb_v, [idx_reg + 3], wdb * inv4s * wposf)
            plsc.store_scatter(regb_v, [idx_reg + 4], soft)
            plsc.store_scatter(regb_v, [idx_reg + 5], wposf)
            return carry

        lax.fori_loop(0, _JMAX, chunk_body, 0)

        pltpu.sync_copy(clsb_v, cls_hbm.at[pl.ds(wid * (_JMAX * _CLS_W),
                                                 _JMAX * _CLS_W)])
        pltpu.sync_copy(regb_v, reg_hbm.at[pl.ds(wid * (_JMAX * _REG_W),
                                                 _JMAX * _REG_W)])


@jax.jit
def kernel(gt_boxes, feature_select_weight, feature_maps_shape):
    packed = jnp.concatenate([
        jnp.transpose(gt_boxes).astype(jnp.float32),
        jnp.transpose(feature_select_weight).astype(jnp.float32),
        jnp.pad(feature_maps_shape.reshape(1, -1).astype(jnp.float32),
                ((0, 0), (0, 22))),
    ])

    mesh = plsc.VectorSubcoreMesh(core_axis_name="c", subcore_axis_name="s")
    run = pl.kernel(
        _tec_body,
        out_type=[
            jax.ShapeDtypeStruct((_NLOC * _CLS_C,), jnp.float32),
            jax.ShapeDtypeStruct((_NLOC * _REG_C,), jnp.float32),
        ],
        mesh=mesh,
        compiler_params=pltpu.CompilerParams(needs_layout_passes=False),
        scratch_types=[
            pltpu.VMEM((11, _NB), jnp.float32),
            pltpu.VMEM((_JMAX * _CLS_W,), jnp.float32),
            pltpu.VMEM((_JMAX * _REG_W,), jnp.float32),
        ],
    )
    cls_flat, reg_flat = run(packed)
    return cls_flat.reshape(_NLOC, _CLS_C), reg_flat.reshape(_NLOC, _REG_C)
